# baseline (speedup 1.0000x reference)
import functools

import jax
import jax.numpy as jnp
from jax import lax
from jax.experimental import pallas as pl
from jax.experimental.pallas import tpu as pltpu
from jax.experimental.pallas import tpu_sc as plsc

NC = 2
NS = 16
EB = 112
NSP = 2
SPLIT0 = 0.2


def _round_up(v, m):
    return (v + m - 1) // m * m


def _sc_mesh():
    return plsc.VectorSubcoreMesh(core_axis_name="c", subcore_axis_name="s",
                                  num_cores=NC, num_subcores=NS)


def _make_sc_msg(n_pad, nb0, nb1, d):
    rows_per_tile = n_pad // NS

    hb = EB // NSP

    def body(h_hbm, srcs_hbm, dsts_hbm, out_hbm,
             src_v, dst_v, rows0, rows1, rows2, msg_sh,
             gs0, gs1, gs2, isa, isb):
        c = lax.axis_index("c")
        s = lax.axis_index("s")
        worker = c * NS + s
        nb = jnp.where(c == 0, nb0, nb1)

        zv = jnp.zeros((16,), jnp.float32)

        def zfill(r, _):
            for k in range(d // 16):
                rows0[r, pl.ds(k * 16, 16)] = zv
            return 0
        lax.fori_loop(0, EB, zfill, 0)
        base = s * rows_per_tile

        def zero_body(j, _):
            pltpu.sync_copy(rows0, msg_sh.at[pl.ds(base + j * EB, EB)])
            return 0
        nfull = rows_per_tile // EB
        lax.fori_loop(0, nfull, zero_body, 0)
        rem = rows_per_tile - nfull * EB
        if rem:
            pltpu.sync_copy(rows0.at[pl.ds(0, rem)],
                            msg_sh.at[pl.ds(base + nfull * EB, rem)])

        rows = (rows0, rows1, rows2)
        gsems = (gs0, gs1, gs2)

        def gather(row, buf):
            for p in range(NSP):
                pltpu.async_copy(
                    h_hbm.at[src_v.at[row, pl.ds(p * hb, hb)]],
                    rows[buf].at[pl.ds(p * hb, hb)], gsems[buf])

        def gather_wait(row, buf):
            for p in range(NSP):
                pltpu.make_async_copy(
                    h_hbm.at[src_v.at[row, pl.ds(p * hb, hb)]],
                    rows[buf].at[pl.ds(p * hb, hb)], gsems[buf]).wait()

        pltpu.sync_copy(srcs_hbm.at[worker].at[pl.ds(0, 6)], src_v)
        pltpu.sync_copy(dsts_hbm.at[worker].at[pl.ds(0, 6)], dst_v)
        gather(0, 0)
        gather(1, 1)
        gather(2, 2)
        plsc.subcore_barrier()

        def group_body(g, _):
            for k in range(6):
                jj = g * 6 + k
                kc = k // 2
                buf = k % 3
                row = kc * 2 + (k % 2)
                if k % 2 == 0:
                    @pl.when(jj + 4 < nb)
                    def _():
                        nxt = (g * 6 + k + 4) // 2
                        nsl = ((kc + 2) % 3) * 2
                        pltpu.async_copy(
                            srcs_hbm.at[worker].at[pl.ds(nxt * 2, 2)],
                            src_v.at[pl.ds(nsl, 2)], isa)
                        pltpu.async_copy(
                            dsts_hbm.at[worker].at[pl.ds(nxt * 2, 2)],
                            dst_v.at[pl.ds(nsl, 2)], isb)
                else:
                    @pl.when(jj + 3 < nb)
                    def _():
                        nxt = (g * 6 + k + 3) // 2
                        nsl = ((kc + 2) % 3) * 2
                        pltpu.make_async_copy(
                            srcs_hbm.at[worker].at[pl.ds(nxt * 2, 2)],
                            src_v.at[pl.ds(nsl, 2)], isa).wait()
                        pltpu.make_async_copy(
                            dsts_hbm.at[worker].at[pl.ds(nxt * 2, 2)],
                            dst_v.at[pl.ds(nsl, 2)], isb).wait()
                gather_wait(row, buf)
                pltpu.sync_copy(rows[buf], msg_sh.at[dst_v.at[row]],
                                add=True)

                @pl.when(jj + 3 < nb)
                def _():
                    nrow = (((kc + 1) + (1 if k % 2 else 0)) % 3) * 2 \
                        + ((k + 3) % 2)
                    gather(nrow, buf)
            return 0
        lax.fori_loop(0, nb // 6, group_body, 0)
        plsc.subcore_barrier()

        pltpu.sync_copy(msg_sh.at[pl.ds(base, rows_per_tile)],
                        out_hbm.at[c].at[pl.ds(base, rows_per_tile)])

    return pl.kernel(
        body,
        out_type=jax.ShapeDtypeStruct((NC, n_pad, d), jnp.float32),
        mesh=_sc_mesh(),
        scratch_types=[
            pltpu.VMEM((6, EB), jnp.int32),
            pltpu.VMEM((6, EB), jnp.int32),
            pltpu.VMEM((EB, d), jnp.float32),
            pltpu.VMEM((EB, d), jnp.float32),
            pltpu.VMEM((EB, d), jnp.float32),
            pltpu.VMEM_SHARED((n_pad, d), jnp.float32),
            pltpu.SemaphoreType.DMA,
            pltpu.SemaphoreType.DMA,
            pltpu.SemaphoreType.DMA,
            pltpu.SemaphoreType.DMA,
            pltpu.SemaphoreType.DMA,
        ],
    )


def _make_sc_deg(n_pad, nb, d):
    rows_per_tile = n_pad // NS

    def body(dsts_hbm, out_hbm, dst_v, ones_v, zbuf, deg_sh, sem):
        c = lax.axis_index("c")
        s = lax.axis_index("s")
        worker = c * NS + s

        ov = jnp.full((16,), 1.0, jnp.float32)
        zv = jnp.zeros((16,), jnp.float32)
        for r in range(16):
            for k in range(d // 16):
                zbuf[r, pl.ds(k * 16, 16)] = zv

        def ones_body(r, _):
            for k in range(d // 16):
                ones_v[r, pl.ds(k * 16, 16)] = ov
            return 0
        lax.fori_loop(0, EB, ones_body, 0)
        base = s * rows_per_tile

        def zero_body(j, _):
            pltpu.sync_copy(zbuf, deg_sh.at[pl.ds(base + j * 16, 16)])
            return 0
        nfull = rows_per_tile // 16
        lax.fori_loop(0, nfull, zero_body, 0)
        rem = rows_per_tile - nfull * 16
        if rem:
            pltpu.sync_copy(zbuf.at[pl.ds(0, rem)],
                            deg_sh.at[pl.ds(base + nfull * 16, rem)])
        pltpu.sync_copy(dsts_hbm.at[worker], dst_v)
        plsc.subcore_barrier()

        def edge_body(j, _):
            pltpu.sync_copy(ones_v, deg_sh.at[dst_v.at[j]], add=True)
            return 0
        lax.fori_loop(0, nb, edge_body, 0)
        plsc.subcore_barrier()

        pltpu.sync_copy(deg_sh.at[pl.ds(base, rows_per_tile)],
                        out_hbm.at[c].at[pl.ds(base, rows_per_tile)])

    return pl.kernel(
        body,
        out_type=jax.ShapeDtypeStruct((NC, n_pad, d), jnp.float32),
        mesh=_sc_mesh(),
        scratch_types=[
            pltpu.VMEM((nb, EB), jnp.int32),
            pltpu.VMEM((EB, d), jnp.float32),
            pltpu.VMEM((16, d), jnp.float32),
            pltpu.VMEM_SHARED((n_pad, d), jnp.float32),
            pltpu.SemaphoreType.DMA,
        ],
    )


def _tc_layer_body(m_ref, h_ref, degw_ref, w_ref, b_ref, out_ref):
    m = m_ref[0] + m_ref[1] + h_ref[...]
    deg = degw_ref[0, :, :1] + degw_ref[1, :, :1] + 1.0
    agg = m / deg
    out_ref[...] = jnp.maximum(
        jnp.dot(agg, w_ref[...], preferred_element_type=jnp.float32)
        + b_ref[...], 0.0)


def _tc_layer(m, h, degw, w, b, blk):
    n, d = h.shape
    grid = (n // blk,)
    return pl.pallas_call(
        _tc_layer_body,
        grid=grid,
        in_specs=[
            pl.BlockSpec((NC, blk, d), lambda i: (0, i, 0)),
            pl.BlockSpec((blk, d), lambda i: (i, 0)),
            pl.BlockSpec((NC, blk, 16), lambda i: (0, i, 0)),
            pl.BlockSpec((d, d), lambda i: (0, 0)),
            pl.BlockSpec((1, d), lambda i: (0, 0)),
        ],
        out_specs=pl.BlockSpec((blk, d), lambda i: (i, 0)),
        out_shape=jax.ShapeDtypeStruct((n, d), jnp.float32),
    )(m, h, degw, w, b)


def _tc_layer3_body(g, m_ref, h_ref, degw_ref, w_ref, b_ref, batch_ref,
                    sums_ref, counts_ref):
    i = pl.program_id(0)
    m = m_ref[0] + m_ref[1] + h_ref[...]
    deg = degw_ref[0, :, :1] + degw_ref[1, :, :1] + 1.0
    agg = m / deg
    h_new = jnp.maximum(
        jnp.dot(agg, w_ref[...], preferred_element_type=jnp.float32)
        + b_ref[...], 0.0)
    blk, d = h_new.shape
    gids = lax.broadcasted_iota(jnp.int32, (blk, g), 1)
    onehot = (batch_ref[...] == gids).astype(jnp.float32)

    @pl.when(i == 0)
    def _():
        sums_ref[...] = jnp.zeros_like(sums_ref)
        counts_ref[...] = jnp.zeros_like(counts_ref)
    contract = (((0,), (0,)), ((), ()))
    sums_ref[...] += lax.dot_general(
        onehot, h_new, contract, preferred_element_type=jnp.float32)
    counts_ref[...] += lax.dot_general(
        onehot, jnp.ones((blk, d), jnp.float32), contract,
        preferred_element_type=jnp.float32)


def _tc_layer3(m, h, degw, w, b, batch2d, g, blk):
    n, d = h.shape
    grid = (n // blk,)
    return pl.pallas_call(
        functools.partial(_tc_layer3_body, g),
        grid=grid,
        in_specs=[
            pl.BlockSpec((NC, blk, d), lambda i: (0, i, 0)),
            pl.BlockSpec((blk, d), lambda i: (i, 0)),
            pl.BlockSpec((NC, blk, 16), lambda i: (0, i, 0)),
            pl.BlockSpec((d, d), lambda i: (0, 0)),
            pl.BlockSpec((1, d), lambda i: (0, 0)),
            pl.BlockSpec((blk, 1), lambda i: (i, 0)),
        ],
        out_specs=[
            pl.BlockSpec((g, d), lambda i: (0, 0)),
            pl.BlockSpec((g, d), lambda i: (0, 0)),
        ],
        out_shape=[
            jax.ShapeDtypeStruct((g, d), jnp.float32),
            jax.ShapeDtypeStruct((g, d), jnp.float32),
        ],
    )(m, h, degw, w, b, batch2d)


def _tc_head_body(sums_ref, counts_ref, v1_ref, vb1_ref, v2r_ref, vb2_ref,
                  out_ref):
    pooled = sums_ref[...] / jnp.maximum(counts_ref[...], 1.0)
    hidden = jnp.maximum(
        jnp.dot(pooled, v1_ref[...], preferred_element_type=jnp.float32)
        + vb1_ref[...], 0.0)
    v = lax.dot_general(v2r_ref[...], hidden, (((1,), (1,)), ((), ())),
                        preferred_element_type=jnp.float32)
    out_ref[...] = v + vb2_ref[...]


def _tc_head(sums, counts, v1, vb1, v2r, vb2, g, d):
    return pl.pallas_call(
        _tc_head_body,
        out_shape=jax.ShapeDtypeStruct((1, g), jnp.float32),
    )(sums, counts, v1, vb1, v2r, vb2)


def kernel(x, edge_index, batch, W1, b1, W2, b2, W3, b3, V1, vb1, V2, vb2):
    n, d = x.shape
    e = edge_index.shape[1]
    g = 64
    nw = NC * NS
    nb = _round_up(_round_up(e, nw * EB) // (nw * EB), 6)
    n_pad = _round_up(n + 1, NS * 8)
    e_pad = nw * nb * EB

    src_flat = jnp.concatenate(
        [edge_index[0], jnp.zeros((e_pad - e,), jnp.int32)])
    dst_flat = jnp.concatenate(
        [edge_index[1], jnp.full((e_pad - e,), n, jnp.int32)])
    dst = dst_flat.reshape(nw, nb, EB)

    nb0 = _round_up(int(round(2 * nb * SPLIT0 / 6.0)) * 6, 6)
    nb1 = 2 * nb - nb0
    cut = NS * nb0 * EB
    s0 = src_flat[:cut].reshape(NS, nb0, EB)
    d0 = dst_flat[:cut].reshape(NS, nb0, EB)
    padb = jnp.zeros((NS, nb1 - nb0, EB), jnp.int32)
    s0 = jnp.concatenate([s0, padb], axis=1)
    d0 = jnp.concatenate([d0, padb + n], axis=1)
    s1 = src_flat[cut:].reshape(NS, nb1, EB)
    d1 = dst_flat[cut:].reshape(NS, nb1, EB)
    src_a = jnp.concatenate([s0, s1], axis=0)
    dst_a = jnp.concatenate([d0, d1], axis=0)

    sc_msg = _make_sc_msg(n_pad, nb0, nb1, d)
    sc_deg = _make_sc_deg(n_pad, nb, d)

    degw = sc_deg(dst)[:, :, :16]

    blk = 1000
    b1r = b1.reshape(1, d)
    b2r = b2.reshape(1, d)
    b3r = b3.reshape(1, d)
    batch2d = batch.reshape(n, 1)

    m1 = sc_msg(x, src_a, dst_a)
    h1 = _tc_layer(m1, x, degw, W1, b1r, blk)
    m2 = sc_msg(h1, src_a, dst_a)
    h2 = _tc_layer(m2, h1, degw, W2, b2r, blk)
    m3 = sc_msg(h2, src_a, dst_a)
    sums, counts = _tc_layer3(m3, h2, degw, W3, b3r, batch2d, g, blk)

    v = _tc_head(sums, counts, V1, vb1.reshape(1, d),
                 V2.reshape(1, d), vb2.reshape(1, 1), g, d)
    return v.reshape(g)


# speedup vs baseline: 1.1780x; 1.1780x over previous
import functools

import jax
import jax.numpy as jnp
from jax import lax
from jax.experimental import pallas as pl
from jax.experimental.pallas import tpu as pltpu
from jax.experimental.pallas import tpu_sc as plsc

NC = 2
NS = 16
EB = 112
NSP = 2
SPLIT0 = 0.68


def _round_up(v, m):
    return (v + m - 1) // m * m


def _sc_mesh():
    return plsc.VectorSubcoreMesh(core_axis_name="c", subcore_axis_name="s",
                                  num_cores=NC, num_subcores=NS)


def _make_sc_msg(n_pad, nb0, nb1, d):
    rows_per_tile = n_pad // NS

    hb = EB // NSP

    def body(h_hbm, srcs_hbm, dsts_hbm, out_hbm,
             src_v, dst_v, rows0, rows1, rows2, msg_sh,
             gs0, gs1, gs2, isa, isb):
        c = lax.axis_index("c")
        s = lax.axis_index("s")
        worker = c * NS + s
        nb = jnp.where(c == 0, nb0, nb1)

        zv = jnp.zeros((16,), jnp.float32)

        def zfill(r, _):
            for k in range(d // 16):
                rows0[r, pl.ds(k * 16, 16)] = zv
            return 0
        lax.fori_loop(0, EB, zfill, 0)
        base = s * rows_per_tile

        def zero_body(j, _):
            pltpu.sync_copy(rows0, msg_sh.at[pl.ds(base + j * EB, EB)])
            return 0
        nfull = rows_per_tile // EB
        lax.fori_loop(0, nfull, zero_body, 0)
        rem = rows_per_tile - nfull * EB
        if rem:
            pltpu.sync_copy(rows0.at[pl.ds(0, rem)],
                            msg_sh.at[pl.ds(base + nfull * EB, rem)])

        rows = (rows0, rows1, rows2)
        gsems = (gs0, gs1, gs2)

        def gather(row, buf):
            for p in range(NSP):
                pltpu.async_copy(
                    h_hbm.at[src_v.at[row, pl.ds(p * hb, hb)]],
                    rows[buf].at[pl.ds(p * hb, hb)], gsems[buf])

        def gather_wait(row, buf):
            for p in range(NSP):
                pltpu.make_async_copy(
                    h_hbm.at[src_v.at[row, pl.ds(p * hb, hb)]],
                    rows[buf].at[pl.ds(p * hb, hb)], gsems[buf]).wait()

        pltpu.sync_copy(srcs_hbm.at[worker].at[pl.ds(0, 6)], src_v)
        pltpu.sync_copy(dsts_hbm.at[worker].at[pl.ds(0, 6)], dst_v)
        gather(0, 0)
        gather(1, 1)
        gather(2, 2)
        plsc.subcore_barrier()

        def group_body(g, _):
            for k in range(6):
                jj = g * 6 + k
                kc = k // 2
                buf = k % 3
                row = kc * 2 + (k % 2)
                if k % 2 == 0:
                    @pl.when(jj + 4 < nb)
                    def _():
                        nxt = (g * 6 + k + 4) // 2
                        nsl = ((kc + 2) % 3) * 2
                        pltpu.async_copy(
                            srcs_hbm.at[worker].at[pl.ds(nxt * 2, 2)],
                            src_v.at[pl.ds(nsl, 2)], isa)
                        pltpu.async_copy(
                            dsts_hbm.at[worker].at[pl.ds(nxt * 2, 2)],
                            dst_v.at[pl.ds(nsl, 2)], isb)
                else:
                    @pl.when(jj + 3 < nb)
                    def _():
                        nxt = (g * 6 + k + 3) // 2
                        nsl = ((kc + 2) % 3) * 2
                        pltpu.make_async_copy(
                            srcs_hbm.at[worker].at[pl.ds(nxt * 2, 2)],
                            src_v.at[pl.ds(nsl, 2)], isa).wait()
                        pltpu.make_async_copy(
                            dsts_hbm.at[worker].at[pl.ds(nxt * 2, 2)],
                            dst_v.at[pl.ds(nsl, 2)], isb).wait()
                gather_wait(row, buf)
                pltpu.sync_copy(rows[buf], msg_sh.at[dst_v.at[row]],
                                add=True)

                @pl.when(jj + 3 < nb)
                def _():
                    nrow = (((kc + 1) + (1 if k % 2 else 0)) % 3) * 2 \
                        + ((k + 3) % 2)
                    gather(nrow, buf)
            return 0
        lax.fori_loop(0, nb // 6, group_body, 0)
        plsc.subcore_barrier()

        pltpu.sync_copy(msg_sh.at[pl.ds(base, rows_per_tile)],
                        out_hbm.at[c].at[pl.ds(base, rows_per_tile)])

    return pl.kernel(
        body,
        out_type=jax.ShapeDtypeStruct((NC, n_pad, d), jnp.float32),
        mesh=_sc_mesh(),
        scratch_types=[
            pltpu.VMEM((6, EB), jnp.int32),
            pltpu.VMEM((6, EB), jnp.int32),
            pltpu.VMEM((EB, d), jnp.float32),
            pltpu.VMEM((EB, d), jnp.float32),
            pltpu.VMEM((EB, d), jnp.float32),
            pltpu.VMEM_SHARED((n_pad, d), jnp.float32),
            pltpu.SemaphoreType.DMA,
            pltpu.SemaphoreType.DMA,
            pltpu.SemaphoreType.DMA,
            pltpu.SemaphoreType.DMA,
            pltpu.SemaphoreType.DMA,
        ],
    )


def _make_sc_deg(n_pad, nb, d):
    rows_per_tile = n_pad // NS

    def body(dsts_hbm, out_hbm, dst_v, ones_v, zbuf, deg_sh, sem):
        c = lax.axis_index("c")
        s = lax.axis_index("s")
        worker = c * NS + s

        ov = jnp.full((16,), 1.0, jnp.float32)
        zv = jnp.zeros((16,), jnp.float32)
        for r in range(16):
            for k in range(d // 16):
                zbuf[r, pl.ds(k * 16, 16)] = zv

        def ones_body(r, _):
            for k in range(d // 16):
                ones_v[r, pl.ds(k * 16, 16)] = ov
            return 0
        lax.fori_loop(0, EB, ones_body, 0)
        base = s * rows_per_tile

        def zero_body(j, _):
            pltpu.sync_copy(zbuf, deg_sh.at[pl.ds(base + j * 16, 16)])
            return 0
        nfull = rows_per_tile // 16
        lax.fori_loop(0, nfull, zero_body, 0)
        rem = rows_per_tile - nfull * 16
        if rem:
            pltpu.sync_copy(zbuf.at[pl.ds(0, rem)],
                            deg_sh.at[pl.ds(base + nfull * 16, rem)])
        pltpu.sync_copy(dsts_hbm.at[worker], dst_v)
        plsc.subcore_barrier()

        def edge_body(j, _):
            pltpu.sync_copy(ones_v, deg_sh.at[dst_v.at[j]], add=True)
            return 0
        lax.fori_loop(0, nb, edge_body, 0)
        plsc.subcore_barrier()

        pltpu.sync_copy(deg_sh.at[pl.ds(base, rows_per_tile)],
                        out_hbm.at[c].at[pl.ds(base, rows_per_tile)])

    return pl.kernel(
        body,
        out_type=jax.ShapeDtypeStruct((NC, n_pad, d), jnp.float32),
        mesh=_sc_mesh(),
        scratch_types=[
            pltpu.VMEM((nb, EB), jnp.int32),
            pltpu.VMEM((EB, d), jnp.float32),
            pltpu.VMEM((16, d), jnp.float32),
            pltpu.VMEM_SHARED((n_pad, d), jnp.float32),
            pltpu.SemaphoreType.DMA,
        ],
    )


def _tc_layer_body(m_ref, h_ref, degw_ref, w_ref, b_ref, out_ref):
    m = m_ref[0] + m_ref[1] + h_ref[...]
    deg = degw_ref[0, :, :1] + degw_ref[1, :, :1] + 1.0
    agg = m / deg
    out_ref[...] = jnp.maximum(
        jnp.dot(agg, w_ref[...], preferred_element_type=jnp.float32)
        + b_ref[...], 0.0)


def _tc_layer(m, h, degw, w, b, blk):
    n, d = h.shape
    grid = (n // blk,)
    return pl.pallas_call(
        _tc_layer_body,
        grid=grid,
        in_specs=[
            pl.BlockSpec((NC, blk, d), lambda i: (0, i, 0)),
            pl.BlockSpec((blk, d), lambda i: (i, 0)),
            pl.BlockSpec((NC, blk, 16), lambda i: (0, i, 0)),
            pl.BlockSpec((d, d), lambda i: (0, 0)),
            pl.BlockSpec((1, d), lambda i: (0, 0)),
        ],
        out_specs=pl.BlockSpec((blk, d), lambda i: (i, 0)),
        out_shape=jax.ShapeDtypeStruct((n, d), jnp.float32),
    )(m, h, degw, w, b)


def _tc_layer3_body(g, m_ref, h_ref, degw_ref, w_ref, b_ref, batch_ref,
                    sums_ref, counts_ref):
    i = pl.program_id(0)
    m = m_ref[0] + m_ref[1] + h_ref[...]
    deg = degw_ref[0, :, :1] + degw_ref[1, :, :1] + 1.0
    agg = m / deg
    h_new = jnp.maximum(
        jnp.dot(agg, w_ref[...], preferred_element_type=jnp.float32)
        + b_ref[...], 0.0)
    blk, d = h_new.shape
    gids = lax.broadcasted_iota(jnp.int32, (blk, g), 1)
    onehot = (batch_ref[...] == gids).astype(jnp.float32)

    @pl.when(i == 0)
    def _():
        sums_ref[...] = jnp.zeros_like(sums_ref)
        counts_ref[...] = jnp.zeros_like(counts_ref)
    contract = (((0,), (0,)), ((), ()))
    sums_ref[...] += lax.dot_general(
        onehot, h_new, contract, preferred_element_type=jnp.float32)
    counts_ref[...] += lax.dot_general(
        onehot, jnp.ones((blk, d), jnp.float32), contract,
        preferred_element_type=jnp.float32)


def _tc_layer3(m, h, degw, w, b, batch2d, g, blk):
    n, d = h.shape
    grid = (n // blk,)
    return pl.pallas_call(
        functools.partial(_tc_layer3_body, g),
        grid=grid,
        in_specs=[
            pl.BlockSpec((NC, blk, d), lambda i: (0, i, 0)),
            pl.BlockSpec((blk, d), lambda i: (i, 0)),
            pl.BlockSpec((NC, blk, 16), lambda i: (0, i, 0)),
            pl.BlockSpec((d, d), lambda i: (0, 0)),
            pl.BlockSpec((1, d), lambda i: (0, 0)),
            pl.BlockSpec((blk, 1), lambda i: (i, 0)),
        ],
        out_specs=[
            pl.BlockSpec((g, d), lambda i: (0, 0)),
            pl.BlockSpec((g, d), lambda i: (0, 0)),
        ],
        out_shape=[
            jax.ShapeDtypeStruct((g, d), jnp.float32),
            jax.ShapeDtypeStruct((g, d), jnp.float32),
        ],
    )(m, h, degw, w, b, batch2d)


def _tc_head_body(sums_ref, counts_ref, v1_ref, vb1_ref, v2r_ref, vb2_ref,
                  out_ref):
    pooled = sums_ref[...] / jnp.maximum(counts_ref[...], 1.0)
    hidden = jnp.maximum(
        jnp.dot(pooled, v1_ref[...], preferred_element_type=jnp.float32)
        + vb1_ref[...], 0.0)
    v = lax.dot_general(v2r_ref[...], hidden, (((1,), (1,)), ((), ())),
                        preferred_element_type=jnp.float32)
    out_ref[...] = v + vb2_ref[...]


def _tc_head(sums, counts, v1, vb1, v2r, vb2, g, d):
    return pl.pallas_call(
        _tc_head_body,
        out_shape=jax.ShapeDtypeStruct((1, g), jnp.float32),
    )(sums, counts, v1, vb1, v2r, vb2)


def kernel(x, edge_index, batch, W1, b1, W2, b2, W3, b3, V1, vb1, V2, vb2):
    n, d = x.shape
    e = edge_index.shape[1]
    g = 64
    nw = NC * NS
    nb = _round_up(_round_up(e, nw * EB) // (nw * EB), 6)
    n_pad = _round_up(n + 1, NS * 8)
    e_pad = nw * nb * EB

    src_flat = jnp.concatenate(
        [edge_index[0], jnp.zeros((e_pad - e,), jnp.int32)])
    dst_flat = jnp.concatenate(
        [edge_index[1], jnp.full((e_pad - e,), n, jnp.int32)])
    dst = dst_flat.reshape(nw, nb, EB)

    nb0 = int(round(2 * nb * SPLIT0 / 6.0)) * 6
    nb1 = 2 * nb - nb0
    nbm = max(nb0, nb1)
    cut = NS * nb0 * EB
    s0 = src_flat[:cut].reshape(NS, nb0, EB)
    d0 = dst_flat[:cut].reshape(NS, nb0, EB)
    s1 = src_flat[cut:].reshape(NS, nb1, EB)
    d1 = dst_flat[cut:].reshape(NS, nb1, EB)
    if nb0 < nbm:
        padb = jnp.zeros((NS, nbm - nb0, EB), jnp.int32)
        s0 = jnp.concatenate([s0, padb], axis=1)
        d0 = jnp.concatenate([d0, padb + n], axis=1)
    if nb1 < nbm:
        padb = jnp.zeros((NS, nbm - nb1, EB), jnp.int32)
        s1 = jnp.concatenate([s1, padb], axis=1)
        d1 = jnp.concatenate([d1, padb + n], axis=1)
    src_a = jnp.concatenate([s0, s1], axis=0)
    dst_a = jnp.concatenate([d0, d1], axis=0)

    sc_msg = _make_sc_msg(n_pad, nb0, nb1, d)
    sc_deg = _make_sc_deg(n_pad, nb, d)

    degw = sc_deg(dst)[:, :, :16]

    blk = 1000
    b1r = b1.reshape(1, d)
    b2r = b2.reshape(1, d)
    b3r = b3.reshape(1, d)
    batch2d = batch.reshape(n, 1)

    m1 = sc_msg(x, src_a, dst_a)
    h1 = _tc_layer(m1, x, degw, W1, b1r, blk)
    m2 = sc_msg(h1, src_a, dst_a)
    h2 = _tc_layer(m2, h1, degw, W2, b2r, blk)
    m3 = sc_msg(h2, src_a, dst_a)
    sums, counts = _tc_layer3(m3, h2, degw, W3, b3r, batch2d, g, blk)

    v = _tc_head(sums, counts, V1, vb1.reshape(1, d),
                 V2.reshape(1, d), vb2.reshape(1, 1), g, d)
    return v.reshape(g)


# speedup vs baseline: 1.2256x; 1.0404x over previous
import functools

import jax
import jax.numpy as jnp
from jax import lax
from jax.experimental import pallas as pl
from jax.experimental.pallas import tpu as pltpu
from jax.experimental.pallas import tpu_sc as plsc

NC = 2
NS = 16
EB = 112
NSP = 2
SPLIT0 = 0.78


def _round_up(v, m):
    return (v + m - 1) // m * m


def _sc_mesh():
    return plsc.VectorSubcoreMesh(core_axis_name="c", subcore_axis_name="s",
                                  num_cores=NC, num_subcores=NS)


def _make_sc_msg(n_pad, nb0, nb1, d):
    rows_per_tile = n_pad // NS

    hb = EB // NSP

    def body(h_hbm, srcs_hbm, dsts_hbm, out_hbm,
             src_v, dst_v, rows0, rows1, rows2, msg_sh,
             gs0, gs1, gs2, isa, isb):
        c = lax.axis_index("c")
        s = lax.axis_index("s")
        worker = c * NS + s
        nb = jnp.where(c == 0, nb0, nb1)

        zv = jnp.zeros((16,), jnp.float32)

        def zfill(r, _):
            for k in range(d // 16):
                rows0[r, pl.ds(k * 16, 16)] = zv
            return 0
        lax.fori_loop(0, EB, zfill, 0)
        base = s * rows_per_tile

        def zero_body(j, _):
            pltpu.sync_copy(rows0, msg_sh.at[pl.ds(base + j * EB, EB)])
            return 0
        nfull = rows_per_tile // EB
        lax.fori_loop(0, nfull, zero_body, 0)
        rem = rows_per_tile - nfull * EB
        if rem:
            pltpu.sync_copy(rows0.at[pl.ds(0, rem)],
                            msg_sh.at[pl.ds(base + nfull * EB, rem)])

        rows = (rows0, rows1, rows2)
        gsems = (gs0, gs1, gs2)

        def gather(row, buf):
            for p in range(NSP):
                pltpu.async_copy(
                    h_hbm.at[src_v.at[row, pl.ds(p * hb, hb)]],
                    rows[buf].at[pl.ds(p * hb, hb)], gsems[buf])

        def gather_wait(row, buf):
            for p in range(NSP):
                pltpu.make_async_copy(
                    h_hbm.at[src_v.at[row, pl.ds(p * hb, hb)]],
                    rows[buf].at[pl.ds(p * hb, hb)], gsems[buf]).wait()

        pltpu.sync_copy(srcs_hbm.at[worker].at[pl.ds(0, 6)], src_v)
        pltpu.sync_copy(dsts_hbm.at[worker].at[pl.ds(0, 6)], dst_v)
        gather(0, 0)
        gather(1, 1)
        gather(2, 2)
        plsc.subcore_barrier()

        def group_body(g, _):
            for k in range(6):
                jj = g * 6 + k
                kc = k // 2
                buf = k % 3
                row = kc * 2 + (k % 2)
                if k % 2 == 0:
                    @pl.when(jj + 4 < nb)
                    def _():
                        nxt = (g * 6 + k + 4) // 2
                        nsl = ((kc + 2) % 3) * 2
                        pltpu.async_copy(
                            srcs_hbm.at[worker].at[pl.ds(nxt * 2, 2)],
                            src_v.at[pl.ds(nsl, 2)], isa)
                        pltpu.async_copy(
                            dsts_hbm.at[worker].at[pl.ds(nxt * 2, 2)],
                            dst_v.at[pl.ds(nsl, 2)], isb)
                gather_wait(row, buf)
                pltpu.sync_copy(rows[buf], msg_sh.at[dst_v.at[row]],
                                add=True)

                @pl.when(jj + 3 < nb)
                def _():
                    if k % 2 == 1:
                        nxt = (g * 6 + k + 3) // 2
                        nsl = ((kc + 2) % 3) * 2
                        pltpu.make_async_copy(
                            srcs_hbm.at[worker].at[pl.ds(nxt * 2, 2)],
                            src_v.at[pl.ds(nsl, 2)], isa).wait()
                        pltpu.make_async_copy(
                            dsts_hbm.at[worker].at[pl.ds(nxt * 2, 2)],
                            dst_v.at[pl.ds(nsl, 2)], isb).wait()
                    nrow = (((kc + 1) + (1 if k % 2 else 0)) % 3) * 2 \
                        + ((k + 3) % 2)
                    gather(nrow, buf)
            return 0
        lax.fori_loop(0, nb // 6, group_body, 0)
        plsc.subcore_barrier()

        pltpu.sync_copy(msg_sh.at[pl.ds(base, rows_per_tile)],
                        out_hbm.at[c].at[pl.ds(base, rows_per_tile)])

    return pl.kernel(
        body,
        out_type=jax.ShapeDtypeStruct((NC, n_pad, d), jnp.float32),
        mesh=_sc_mesh(),
        scratch_types=[
            pltpu.VMEM((6, EB), jnp.int32),
            pltpu.VMEM((6, EB), jnp.int32),
            pltpu.VMEM((EB, d), jnp.float32),
            pltpu.VMEM((EB, d), jnp.float32),
            pltpu.VMEM((EB, d), jnp.float32),
            pltpu.VMEM_SHARED((n_pad, d), jnp.float32),
            pltpu.SemaphoreType.DMA,
            pltpu.SemaphoreType.DMA,
            pltpu.SemaphoreType.DMA,
            pltpu.SemaphoreType.DMA,
            pltpu.SemaphoreType.DMA,
        ],
    )


def _make_sc_deg(n_pad, nb, d):
    rows_per_tile = n_pad // NS

    def body(dsts_hbm, out_hbm, dst_v, ones_v, zbuf, deg_sh, sem):
        c = lax.axis_index("c")
        s = lax.axis_index("s")
        worker = c * NS + s

        ov = jnp.full((16,), 1.0, jnp.float32)
        zv = jnp.zeros((16,), jnp.float32)
        for r in range(16):
            for k in range(d // 16):
                zbuf[r, pl.ds(k * 16, 16)] = zv

        def ones_body(r, _):
            for k in range(d // 16):
                ones_v[r, pl.ds(k * 16, 16)] = ov
            return 0
        lax.fori_loop(0, EB, ones_body, 0)
        base = s * rows_per_tile

        def zero_body(j, _):
            pltpu.sync_copy(zbuf, deg_sh.at[pl.ds(base + j * 16, 16)])
            return 0
        nfull = rows_per_tile // 16
        lax.fori_loop(0, nfull, zero_body, 0)
        rem = rows_per_tile - nfull * 16
        if rem:
            pltpu.sync_copy(zbuf.at[pl.ds(0, rem)],
                            deg_sh.at[pl.ds(base + nfull * 16, rem)])
        pltpu.sync_copy(dsts_hbm.at[worker], dst_v)
        plsc.subcore_barrier()

        def edge_body(j, _):
            pltpu.sync_copy(ones_v, deg_sh.at[dst_v.at[j]], add=True)
            return 0
        lax.fori_loop(0, nb, edge_body, 0)
        plsc.subcore_barrier()

        pltpu.sync_copy(deg_sh.at[pl.ds(base, rows_per_tile)],
                        out_hbm.at[c].at[pl.ds(base, rows_per_tile)])

    return pl.kernel(
        body,
        out_type=jax.ShapeDtypeStruct((NC, n_pad, d), jnp.float32),
        mesh=_sc_mesh(),
        scratch_types=[
            pltpu.VMEM((nb, EB), jnp.int32),
            pltpu.VMEM((EB, d), jnp.float32),
            pltpu.VMEM((16, d), jnp.float32),
            pltpu.VMEM_SHARED((n_pad, d), jnp.float32),
            pltpu.SemaphoreType.DMA,
        ],
    )


def _tc_layer_body(m_ref, h_ref, degw_ref, w_ref, b_ref, out_ref):
    m = m_ref[0] + m_ref[1] + h_ref[...]
    deg = degw_ref[0, :, :1] + degw_ref[1, :, :1] + 1.0
    agg = m / deg
    out_ref[...] = jnp.maximum(
        jnp.dot(agg, w_ref[...], preferred_element_type=jnp.float32)
        + b_ref[...], 0.0)


def _tc_layer(m, h, degw, w, b, blk):
    n, d = h.shape
    grid = (n // blk,)
    return pl.pallas_call(
        _tc_layer_body,
        grid=grid,
        in_specs=[
            pl.BlockSpec((NC, blk, d), lambda i: (0, i, 0)),
            pl.BlockSpec((blk, d), lambda i: (i, 0)),
            pl.BlockSpec((NC, blk, 16), lambda i: (0, i, 0)),
            pl.BlockSpec((d, d), lambda i: (0, 0)),
            pl.BlockSpec((1, d), lambda i: (0, 0)),
        ],
        out_specs=pl.BlockSpec((blk, d), lambda i: (i, 0)),
        out_shape=jax.ShapeDtypeStruct((n, d), jnp.float32),
    )(m, h, degw, w, b)


def _tc_layer3_body(g, m_ref, h_ref, degw_ref, w_ref, b_ref, batch_ref,
                    sums_ref, counts_ref):
    i = pl.program_id(0)
    m = m_ref[0] + m_ref[1] + h_ref[...]
    deg = degw_ref[0, :, :1] + degw_ref[1, :, :1] + 1.0
    agg = m / deg
    h_new = jnp.maximum(
        jnp.dot(agg, w_ref[...], preferred_element_type=jnp.float32)
        + b_ref[...], 0.0)
    blk, d = h_new.shape
    gids = lax.broadcasted_iota(jnp.int32, (blk, g), 1)
    onehot = (batch_ref[...] == gids).astype(jnp.float32)

    @pl.when(i == 0)
    def _():
        sums_ref[...] = jnp.zeros_like(sums_ref)
        counts_ref[...] = jnp.zeros_like(counts_ref)
    contract = (((0,), (0,)), ((), ()))
    sums_ref[...] += lax.dot_general(
        onehot, h_new, contract, preferred_element_type=jnp.float32)
    counts_ref[...] += lax.dot_general(
        onehot, jnp.ones((blk, d), jnp.float32), contract,
        preferred_element_type=jnp.float32)


def _tc_layer3(m, h, degw, w, b, batch2d, g, blk):
    n, d = h.shape
    grid = (n // blk,)
    return pl.pallas_call(
        functools.partial(_tc_layer3_body, g),
        grid=grid,
        in_specs=[
            pl.BlockSpec((NC, blk, d), lambda i: (0, i, 0)),
            pl.BlockSpec((blk, d), lambda i: (i, 0)),
            pl.BlockSpec((NC, blk, 16), lambda i: (0, i, 0)),
            pl.BlockSpec((d, d), lambda i: (0, 0)),
            pl.BlockSpec((1, d), lambda i: (0, 0)),
            pl.BlockSpec((blk, 1), lambda i: (i, 0)),
        ],
        out_specs=[
            pl.BlockSpec((g, d), lambda i: (0, 0)),
            pl.BlockSpec((g, d), lambda i: (0, 0)),
        ],
        out_shape=[
            jax.ShapeDtypeStruct((g, d), jnp.float32),
            jax.ShapeDtypeStruct((g, d), jnp.float32),
        ],
    )(m, h, degw, w, b, batch2d)


def _tc_head_body(sums_ref, counts_ref, v1_ref, vb1_ref, v2r_ref, vb2_ref,
                  out_ref):
    pooled = sums_ref[...] / jnp.maximum(counts_ref[...], 1.0)
    hidden = jnp.maximum(
        jnp.dot(pooled, v1_ref[...], preferred_element_type=jnp.float32)
        + vb1_ref[...], 0.0)
    v = lax.dot_general(v2r_ref[...], hidden, (((1,), (1,)), ((), ())),
                        preferred_element_type=jnp.float32)
    out_ref[...] = v + vb2_ref[...]


def _tc_head(sums, counts, v1, vb1, v2r, vb2, g, d):
    return pl.pallas_call(
        _tc_head_body,
        out_shape=jax.ShapeDtypeStruct((1, g), jnp.float32),
    )(sums, counts, v1, vb1, v2r, vb2)


def kernel(x, edge_index, batch, W1, b1, W2, b2, W3, b3, V1, vb1, V2, vb2):
    n, d = x.shape
    e = edge_index.shape[1]
    g = 64
    nw = NC * NS
    nb = _round_up(_round_up(e, nw * EB) // (nw * EB), 6)
    n_pad = _round_up(n + 1, NS * 8)
    e_pad = nw * nb * EB

    src_flat = jnp.concatenate(
        [edge_index[0], jnp.zeros((e_pad - e,), jnp.int32)])
    dst_flat = jnp.concatenate(
        [edge_index[1], jnp.full((e_pad - e,), n, jnp.int32)])
    dst = dst_flat.reshape(nw, nb, EB)

    nb0 = int(round(2 * nb * SPLIT0 / 6.0)) * 6
    nb1 = 2 * nb - nb0
    nbm = max(nb0, nb1)
    cut = NS * nb0 * EB
    s0 = src_flat[:cut].reshape(NS, nb0, EB)
    d0 = dst_flat[:cut].reshape(NS, nb0, EB)
    s1 = src_flat[cut:].reshape(NS, nb1, EB)
    d1 = dst_flat[cut:].reshape(NS, nb1, EB)
    if nb0 < nbm:
        padb = jnp.zeros((NS, nbm - nb0, EB), jnp.int32)
        s0 = jnp.concatenate([s0, padb], axis=1)
        d0 = jnp.concatenate([d0, padb + n], axis=1)
    if nb1 < nbm:
        padb = jnp.zeros((NS, nbm - nb1, EB), jnp.int32)
        s1 = jnp.concatenate([s1, padb], axis=1)
        d1 = jnp.concatenate([d1, padb + n], axis=1)
    src_a = jnp.concatenate([s0, s1], axis=0)
    dst_a = jnp.concatenate([d0, d1], axis=0)

    sc_msg = _make_sc_msg(n_pad, nb0, nb1, d)
    sc_deg = _make_sc_deg(n_pad, nb, d)

    degw = sc_deg(dst)[:, :, :16]

    blk = 1000
    b1r = b1.reshape(1, d)
    b2r = b2.reshape(1, d)
    b3r = b3.reshape(1, d)
    batch2d = batch.reshape(n, 1)

    m1 = sc_msg(x, src_a, dst_a)
    h1 = _tc_layer(m1, x, degw, W1, b1r, blk)
    m2 = sc_msg(h1, src_a, dst_a)
    h2 = _tc_layer(m2, h1, degw, W2, b2r, blk)
    m3 = sc_msg(h2, src_a, dst_a)
    sums, counts = _tc_layer3(m3, h2, degw, W3, b3r, batch2d, g, blk)

    v = _tc_head(sums, counts, V1, vb1.reshape(1, d),
                 V2.reshape(1, d), vb2.reshape(1, 1), g, d)
    return v.reshape(g)


# speedup vs baseline: 1.2761x; 1.0412x over previous
import functools

import jax
import jax.numpy as jnp
from jax import lax
from jax.experimental import pallas as pl
from jax.experimental.pallas import tpu as pltpu
from jax.experimental.pallas import tpu_sc as plsc

NC = 2
NS = 16
EB = 112
NSP = 2
SPLIT0 = 0.86


def _round_up(v, m):
    return (v + m - 1) // m * m


def _sc_mesh():
    return plsc.VectorSubcoreMesh(core_axis_name="c", subcore_axis_name="s",
                                  num_cores=NC, num_subcores=NS)


def _make_sc_msg(n_pad, nb0, nb1, d):
    rows_per_tile = n_pad // NS

    hb = EB // NSP

    def body(h_hbm, srcs_hbm, dsts_hbm, out_hbm,
             src_v, dst_v, rows0, rows1, rows2, msg_sh,
             gs0, gs1, gs2, isa, isb):
        c = lax.axis_index("c")
        s = lax.axis_index("s")
        worker = c * NS + s
        nb = jnp.where(c == 0, nb0, nb1)

        zv = jnp.zeros((16,), jnp.float32)

        def zfill(r, _):
            for k in range(d // 16):
                rows0[r, pl.ds(k * 16, 16)] = zv
            return 0
        lax.fori_loop(0, EB, zfill, 0)
        base = s * rows_per_tile

        def zero_body(j, _):
            pltpu.sync_copy(rows0, msg_sh.at[pl.ds(base + j * EB, EB)])
            return 0
        nfull = rows_per_tile // EB
        lax.fori_loop(0, nfull, zero_body, 0)
        rem = rows_per_tile - nfull * EB
        if rem:
            pltpu.sync_copy(rows0.at[pl.ds(0, rem)],
                            msg_sh.at[pl.ds(base + nfull * EB, rem)])

        rows = (rows0, rows1, rows2)
        gsems = (gs0, gs1, gs2)

        def gather(row, buf):
            for p in range(NSP):
                pltpu.async_copy(
                    h_hbm.at[src_v.at[row, pl.ds(p * hb, hb)]],
                    rows[buf].at[pl.ds(p * hb, hb)], gsems[buf])

        def gather_wait(row, buf):
            for p in range(NSP):
                pltpu.make_async_copy(
                    h_hbm.at[src_v.at[row, pl.ds(p * hb, hb)]],
                    rows[buf].at[pl.ds(p * hb, hb)], gsems[buf]).wait()

        pltpu.sync_copy(srcs_hbm.at[worker].at[pl.ds(0, 6)], src_v)
        pltpu.sync_copy(dsts_hbm.at[worker].at[pl.ds(0, 6)], dst_v)
        gather(0, 0)
        gather(1, 1)
        gather(2, 2)
        plsc.subcore_barrier()

        def group_body(g, _):
            for k in range(6):
                jj = g * 6 + k
                kc = k // 2
                buf = k % 3
                row = kc * 2 + (k % 2)
                if k % 2 == 0:
                    @pl.when(jj + 4 < nb)
                    def _():
                        nxt = (g * 6 + k + 4) // 2
                        nsl = ((kc + 2) % 3) * 2
                        pltpu.async_copy(
                            srcs_hbm.at[worker].at[pl.ds(nxt * 2, 2)],
                            src_v.at[pl.ds(nsl, 2)], isa)
                        pltpu.async_copy(
                            dsts_hbm.at[worker].at[pl.ds(nxt * 2, 2)],
                            dst_v.at[pl.ds(nsl, 2)], isb)
                gather_wait(row, buf)
                pltpu.sync_copy(rows[buf], msg_sh.at[dst_v.at[row]],
                                add=True)

                @pl.when(jj + 3 < nb)
                def _():
                    if k % 2 == 1:
                        nxt = (g * 6 + k + 3) // 2
                        nsl = ((kc + 2) % 3) * 2
                        pltpu.make_async_copy(
                            srcs_hbm.at[worker].at[pl.ds(nxt * 2, 2)],
                            src_v.at[pl.ds(nsl, 2)], isa).wait()
                        pltpu.make_async_copy(
                            dsts_hbm.at[worker].at[pl.ds(nxt * 2, 2)],
                            dst_v.at[pl.ds(nsl, 2)], isb).wait()
                    nrow = (((kc + 1) + (1 if k % 2 else 0)) % 3) * 2 \
                        + ((k + 3) % 2)
                    gather(nrow, buf)
            return 0
        lax.fori_loop(0, nb // 6, group_body, 0)
        plsc.subcore_barrier()

        pltpu.sync_copy(msg_sh.at[pl.ds(base, rows_per_tile)],
                        out_hbm.at[c].at[pl.ds(base, rows_per_tile)])

    return pl.kernel(
        body,
        out_type=jax.ShapeDtypeStruct((NC, n_pad, d), jnp.float32),
        mesh=_sc_mesh(),
        scratch_types=[
            pltpu.VMEM((6, EB), jnp.int32),
            pltpu.VMEM((6, EB), jnp.int32),
            pltpu.VMEM((EB, d), jnp.float32),
            pltpu.VMEM((EB, d), jnp.float32),
            pltpu.VMEM((EB, d), jnp.float32),
            pltpu.VMEM_SHARED((n_pad, d), jnp.float32),
            pltpu.SemaphoreType.DMA,
            pltpu.SemaphoreType.DMA,
            pltpu.SemaphoreType.DMA,
            pltpu.SemaphoreType.DMA,
            pltpu.SemaphoreType.DMA,
        ],
    )


def _make_sc_deg(n_pad, nb, d):
    rows_per_tile = n_pad // NS

    def body(dsts_hbm, out_hbm, dst_v, ones_v, zbuf, deg_sh, sem):
        c = lax.axis_index("c")
        s = lax.axis_index("s")
        worker = c * NS + s

        ov = jnp.full((16,), 1.0, jnp.float32)
        zv = jnp.zeros((16,), jnp.float32)
        for r in range(16):
            for k in range(d // 16):
                zbuf[r, pl.ds(k * 16, 16)] = zv

        def ones_body(r, _):
            for k in range(d // 16):
                ones_v[r, pl.ds(k * 16, 16)] = ov
            return 0
        lax.fori_loop(0, EB, ones_body, 0)
        base = s * rows_per_tile

        def zero_body(j, _):
            pltpu.sync_copy(zbuf, deg_sh.at[pl.ds(base + j * 16, 16)])
            return 0
        nfull = rows_per_tile // 16
        lax.fori_loop(0, nfull, zero_body, 0)
        rem = rows_per_tile - nfull * 16
        if rem:
            pltpu.sync_copy(zbuf.at[pl.ds(0, rem)],
                            deg_sh.at[pl.ds(base + nfull * 16, rem)])
        pltpu.sync_copy(dsts_hbm.at[worker], dst_v)
        plsc.subcore_barrier()

        def edge_body(j, _):
            pltpu.sync_copy(ones_v, deg_sh.at[dst_v.at[j]], add=True)
            return 0
        lax.fori_loop(0, nb, edge_body, 0)
        plsc.subcore_barrier()

        pltpu.sync_copy(deg_sh.at[pl.ds(base, rows_per_tile)],
                        out_hbm.at[c].at[pl.ds(base, rows_per_tile)])

    return pl.kernel(
        body,
        out_type=jax.ShapeDtypeStruct((NC, n_pad, d), jnp.float32),
        mesh=_sc_mesh(),
        scratch_types=[
            pltpu.VMEM((nb, EB), jnp.int32),
            pltpu.VMEM((EB, d), jnp.float32),
            pltpu.VMEM((16, d), jnp.float32),
            pltpu.VMEM_SHARED((n_pad, d), jnp.float32),
            pltpu.SemaphoreType.DMA,
        ],
    )


def _tc_layer_body(m_ref, h_ref, degw_ref, w_ref, b_ref, out_ref):
    m = m_ref[0] + m_ref[1] + h_ref[...]
    deg = degw_ref[0, :, :1] + degw_ref[1, :, :1] + 1.0
    agg = m / deg
    out_ref[...] = jnp.maximum(
        jnp.dot(agg, w_ref[...], preferred_element_type=jnp.float32)
        + b_ref[...], 0.0)


def _tc_layer(m, h, degw, w, b, blk):
    n, d = h.shape
    grid = (n // blk,)
    return pl.pallas_call(
        _tc_layer_body,
        grid=grid,
        in_specs=[
            pl.BlockSpec((NC, blk, d), lambda i: (0, i, 0)),
            pl.BlockSpec((blk, d), lambda i: (i, 0)),
            pl.BlockSpec((NC, blk, 16), lambda i: (0, i, 0)),
            pl.BlockSpec((d, d), lambda i: (0, 0)),
            pl.BlockSpec((1, d), lambda i: (0, 0)),
        ],
        out_specs=pl.BlockSpec((blk, d), lambda i: (i, 0)),
        out_shape=jax.ShapeDtypeStruct((n, d), jnp.float32),
    )(m, h, degw, w, b)


def _tc_layer3_body(g, m_ref, h_ref, degw_ref, w_ref, b_ref, batch_ref,
                    sums_ref, counts_ref):
    i = pl.program_id(0)
    m = m_ref[0] + m_ref[1] + h_ref[...]
    deg = degw_ref[0, :, :1] + degw_ref[1, :, :1] + 1.0
    agg = m / deg
    h_new = jnp.maximum(
        jnp.dot(agg, w_ref[...], preferred_element_type=jnp.float32)
        + b_ref[...], 0.0)
    blk, d = h_new.shape
    gids = lax.broadcasted_iota(jnp.int32, (blk, g), 1)
    onehot = (batch_ref[...] == gids).astype(jnp.float32)

    @pl.when(i == 0)
    def _():
        sums_ref[...] = jnp.zeros_like(sums_ref)
        counts_ref[...] = jnp.zeros_like(counts_ref)
    contract = (((0,), (0,)), ((), ()))
    sums_ref[...] += lax.dot_general(
        onehot, h_new, contract, preferred_element_type=jnp.float32)
    counts_ref[...] += lax.dot_general(
        onehot, jnp.ones((blk, d), jnp.float32), contract,
        preferred_element_type=jnp.float32)


def _tc_layer3(m, h, degw, w, b, batch2d, g, blk):
    n, d = h.shape
    grid = (n // blk,)
    return pl.pallas_call(
        functools.partial(_tc_layer3_body, g),
        grid=grid,
        in_specs=[
            pl.BlockSpec((NC, blk, d), lambda i: (0, i, 0)),
            pl.BlockSpec((blk, d), lambda i: (i, 0)),
            pl.BlockSpec((NC, blk, 16), lambda i: (0, i, 0)),
            pl.BlockSpec((d, d), lambda i: (0, 0)),
            pl.BlockSpec((1, d), lambda i: (0, 0)),
            pl.BlockSpec((blk, 1), lambda i: (i, 0)),
        ],
        out_specs=[
            pl.BlockSpec((g, d), lambda i: (0, 0)),
            pl.BlockSpec((g, d), lambda i: (0, 0)),
        ],
        out_shape=[
            jax.ShapeDtypeStruct((g, d), jnp.float32),
            jax.ShapeDtypeStruct((g, d), jnp.float32),
        ],
    )(m, h, degw, w, b, batch2d)


def _tc_head_body(sums_ref, counts_ref, v1_ref, vb1_ref, v2r_ref, vb2_ref,
                  out_ref):
    pooled = sums_ref[...] / jnp.maximum(counts_ref[...], 1.0)
    hidden = jnp.maximum(
        jnp.dot(pooled, v1_ref[...], preferred_element_type=jnp.float32)
        + vb1_ref[...], 0.0)
    v = lax.dot_general(v2r_ref[...], hidden, (((1,), (1,)), ((), ())),
                        preferred_element_type=jnp.float32)
    out_ref[...] = v + vb2_ref[...]


def _tc_head(sums, counts, v1, vb1, v2r, vb2, g, d):
    return pl.pallas_call(
        _tc_head_body,
        out_shape=jax.ShapeDtypeStruct((1, g), jnp.float32),
    )(sums, counts, v1, vb1, v2r, vb2)


def kernel(x, edge_index, batch, W1, b1, W2, b2, W3, b3, V1, vb1, V2, vb2):
    n, d = x.shape
    e = edge_index.shape[1]
    g = 64
    nw = NC * NS
    nb = _round_up(_round_up(e, nw * EB) // (nw * EB), 6)
    n_pad = _round_up(n + 1, NS * 8)
    e_pad = nw * nb * EB

    src_flat = jnp.concatenate(
        [edge_index[0], jnp.zeros((e_pad - e,), jnp.int32)])
    dst_flat = jnp.concatenate(
        [edge_index[1], jnp.full((e_pad - e,), n, jnp.int32)])
    dst = dst_flat.reshape(nw, nb, EB)

    nb0 = int(round(2 * nb * SPLIT0 / 6.0)) * 6
    nb1 = 2 * nb - nb0
    nbm = max(nb0, nb1)
    cut = NS * nb0 * EB
    s0 = src_flat[:cut].reshape(NS, nb0, EB)
    d0 = dst_flat[:cut].reshape(NS, nb0, EB)
    s1 = src_flat[cut:].reshape(NS, nb1, EB)
    d1 = dst_flat[cut:].reshape(NS, nb1, EB)
    if nb0 < nbm:
        padb = jnp.zeros((NS, nbm - nb0, EB), jnp.int32)
        s0 = jnp.concatenate([s0, padb], axis=1)
        d0 = jnp.concatenate([d0, padb + n], axis=1)
    if nb1 < nbm:
        padb = jnp.zeros((NS, nbm - nb1, EB), jnp.int32)
        s1 = jnp.concatenate([s1, padb], axis=1)
        d1 = jnp.concatenate([d1, padb + n], axis=1)
    src_a = jnp.concatenate([s0, s1], axis=0)
    dst_a = jnp.concatenate([d0, d1], axis=0)

    sc_msg = _make_sc_msg(n_pad, nb0, nb1, d)
    sc_deg = _make_sc_deg(n_pad, nb, d)

    degw = sc_deg(dst)[:, :, :16]

    blk = 1000
    b1r = b1.reshape(1, d)
    b2r = b2.reshape(1, d)
    b3r = b3.reshape(1, d)
    batch2d = batch.reshape(n, 1)

    m1 = sc_msg(x, src_a, dst_a)
    h1 = _tc_layer(m1, x, degw, W1, b1r, blk)
    m2 = sc_msg(h1, src_a, dst_a)
    h2 = _tc_layer(m2, h1, degw, W2, b2r, blk)
    m3 = sc_msg(h2, src_a, dst_a)
    sums, counts = _tc_layer3(m3, h2, degw, W3, b3r, batch2d, g, blk)

    v = _tc_head(sums, counts, V1, vb1.reshape(1, d),
                 V2.reshape(1, d), vb2.reshape(1, 1), g, d)
    return v.reshape(g)


# speedup vs baseline: 1.2939x; 1.0140x over previous
import functools

import jax
import jax.numpy as jnp
from jax import lax
from jax.experimental import pallas as pl
from jax.experimental.pallas import tpu as pltpu
from jax.experimental.pallas import tpu_sc as plsc

NC = 2
NS = 16
EB = 112
NSP = 2
SPLIT0 = 0.93


def _round_up(v, m):
    return (v + m - 1) // m * m


def _sc_mesh():
    return plsc.VectorSubcoreMesh(core_axis_name="c", subcore_axis_name="s",
                                  num_cores=NC, num_subcores=NS)


def _make_sc_msg(n_pad, nb0, nb1, d):
    rows_per_tile = n_pad // NS

    hb = EB // NSP

    def body(h_hbm, srcs_hbm, dsts_hbm, out_hbm,
             src_v, dst_v, rows0, rows1, rows2, msg_sh,
             gs0, gs1, gs2, isa, isb):
        c = lax.axis_index("c")
        s = lax.axis_index("s")
        worker = c * NS + s
        nb = jnp.where(c == 0, nb0, nb1)

        zv = jnp.zeros((16,), jnp.float32)

        def zfill(r, _):
            for k in range(d // 16):
                rows0[r, pl.ds(k * 16, 16)] = zv
            return 0
        lax.fori_loop(0, EB, zfill, 0)
        base = s * rows_per_tile

        def zero_body(j, _):
            pltpu.sync_copy(rows0, msg_sh.at[pl.ds(base + j * EB, EB)])
            return 0
        nfull = rows_per_tile // EB
        lax.fori_loop(0, nfull, zero_body, 0)
        rem = rows_per_tile - nfull * EB
        if rem:
            pltpu.sync_copy(rows0.at[pl.ds(0, rem)],
                            msg_sh.at[pl.ds(base + nfull * EB, rem)])

        rows = (rows0, rows1, rows2)
        gsems = (gs0, gs1, gs2)

        def gather(row, buf):
            for p in range(NSP):
                pltpu.async_copy(
                    h_hbm.at[src_v.at[row, pl.ds(p * hb, hb)]],
                    rows[buf].at[pl.ds(p * hb, hb)], gsems[buf])

        def gather_wait(row, buf):
            for p in range(NSP):
                pltpu.make_async_copy(
                    h_hbm.at[src_v.at[row, pl.ds(p * hb, hb)]],
                    rows[buf].at[pl.ds(p * hb, hb)], gsems[buf]).wait()

        pltpu.sync_copy(srcs_hbm.at[worker].at[pl.ds(0, 6)], src_v)
        pltpu.sync_copy(dsts_hbm.at[worker].at[pl.ds(0, 6)], dst_v)
        gather(0, 0)
        gather(1, 1)
        gather(2, 2)
        plsc.subcore_barrier()

        def group_body(g, _):
            for k in range(6):
                jj = g * 6 + k
                kc = k // 2
                buf = k % 3
                row = kc * 2 + (k % 2)
                if k % 2 == 0:
                    @pl.when(jj + 4 < nb)
                    def _():
                        nxt = (g * 6 + k + 4) // 2
                        nsl = ((kc + 2) % 3) * 2
                        pltpu.async_copy(
                            srcs_hbm.at[worker].at[pl.ds(nxt * 2, 2)],
                            src_v.at[pl.ds(nsl, 2)], isa)
                        pltpu.async_copy(
                            dsts_hbm.at[worker].at[pl.ds(nxt * 2, 2)],
                            dst_v.at[pl.ds(nsl, 2)], isb)
                gather_wait(row, buf)
                pltpu.sync_copy(rows[buf], msg_sh.at[dst_v.at[row]],
                                add=True)

                @pl.when(jj + 3 < nb)
                def _():
                    if k % 2 == 1:
                        nxt = (g * 6 + k + 3) // 2
                        nsl = ((kc + 2) % 3) * 2
                        pltpu.make_async_copy(
                            srcs_hbm.at[worker].at[pl.ds(nxt * 2, 2)],
                            src_v.at[pl.ds(nsl, 2)], isa).wait()
                        pltpu.make_async_copy(
                            dsts_hbm.at[worker].at[pl.ds(nxt * 2, 2)],
                            dst_v.at[pl.ds(nsl, 2)], isb).wait()
                    nrow = (((kc + 1) + (1 if k % 2 else 0)) % 3) * 2 \
                        + ((k + 3) % 2)
                    gather(nrow, buf)
            return 0
        lax.fori_loop(0, nb // 6, group_body, 0)
        plsc.subcore_barrier()

        pltpu.sync_copy(msg_sh.at[pl.ds(base, rows_per_tile)],
                        out_hbm.at[c].at[pl.ds(base, rows_per_tile)])

    return pl.kernel(
        body,
        out_type=jax.ShapeDtypeStruct((NC, n_pad, d), jnp.float32),
        mesh=_sc_mesh(),
        scratch_types=[
            pltpu.VMEM((6, EB), jnp.int32),
            pltpu.VMEM((6, EB), jnp.int32),
            pltpu.VMEM((EB, d), jnp.float32),
            pltpu.VMEM((EB, d), jnp.float32),
            pltpu.VMEM((EB, d), jnp.float32),
            pltpu.VMEM_SHARED((n_pad, d), jnp.float32),
            pltpu.SemaphoreType.DMA,
            pltpu.SemaphoreType.DMA,
            pltpu.SemaphoreType.DMA,
            pltpu.SemaphoreType.DMA,
            pltpu.SemaphoreType.DMA,
        ],
    )


def _make_sc_deg(n_pad, nb, d):
    rows_per_tile = n_pad // NS

    def body(dsts_hbm, out_hbm, dst_v, ones_v, zbuf, deg_sh, sem):
        c = lax.axis_index("c")
        s = lax.axis_index("s")
        worker = c * NS + s

        ov = jnp.full((16,), 1.0, jnp.float32)
        zv = jnp.zeros((16,), jnp.float32)
        for r in range(16):
            for k in range(d // 16):
                zbuf[r, pl.ds(k * 16, 16)] = zv

        def ones_body(r, _):
            for k in range(d // 16):
                ones_v[r, pl.ds(k * 16, 16)] = ov
            return 0
        lax.fori_loop(0, EB, ones_body, 0)
        base = s * rows_per_tile

        def zero_body(j, _):
            pltpu.sync_copy(zbuf, deg_sh.at[pl.ds(base + j * 16, 16)])
            return 0
        nfull = rows_per_tile // 16
        lax.fori_loop(0, nfull, zero_body, 0)
        rem = rows_per_tile - nfull * 16
        if rem:
            pltpu.sync_copy(zbuf.at[pl.ds(0, rem)],
                            deg_sh.at[pl.ds(base + nfull * 16, rem)])
        pltpu.sync_copy(dsts_hbm.at[worker], dst_v)
        plsc.subcore_barrier()

        def edge_body(j, _):
            pltpu.sync_copy(ones_v, deg_sh.at[dst_v.at[j]], add=True)
            return 0
        lax.fori_loop(0, nb, edge_body, 0)
        plsc.subcore_barrier()

        pltpu.sync_copy(deg_sh.at[pl.ds(base, rows_per_tile)],
                        out_hbm.at[c].at[pl.ds(base, rows_per_tile)])

    return pl.kernel(
        body,
        out_type=jax.ShapeDtypeStruct((NC, n_pad, d), jnp.float32),
        mesh=_sc_mesh(),
        scratch_types=[
            pltpu.VMEM((nb, EB), jnp.int32),
            pltpu.VMEM((EB, d), jnp.float32),
            pltpu.VMEM((16, d), jnp.float32),
            pltpu.VMEM_SHARED((n_pad, d), jnp.float32),
            pltpu.SemaphoreType.DMA,
        ],
    )


def _tc_layer_body(m_ref, h_ref, degw_ref, w_ref, b_ref, out_ref):
    m = m_ref[0] + m_ref[1] + h_ref[...]
    deg = degw_ref[0, :, :1] + degw_ref[1, :, :1] + 1.0
    agg = m / deg
    out_ref[...] = jnp.maximum(
        jnp.dot(agg, w_ref[...], preferred_element_type=jnp.float32)
        + b_ref[...], 0.0)


def _tc_layer(m, h, degw, w, b, blk):
    n, d = h.shape
    grid = (n // blk,)
    return pl.pallas_call(
        _tc_layer_body,
        grid=grid,
        in_specs=[
            pl.BlockSpec((NC, blk, d), lambda i: (0, i, 0)),
            pl.BlockSpec((blk, d), lambda i: (i, 0)),
            pl.BlockSpec((NC, blk, 16), lambda i: (0, i, 0)),
            pl.BlockSpec((d, d), lambda i: (0, 0)),
            pl.BlockSpec((1, d), lambda i: (0, 0)),
        ],
        out_specs=pl.BlockSpec((blk, d), lambda i: (i, 0)),
        out_shape=jax.ShapeDtypeStruct((n, d), jnp.float32),
    )(m, h, degw, w, b)


def _tc_layer3_body(g, m_ref, h_ref, degw_ref, w_ref, b_ref, batch_ref,
                    sums_ref, counts_ref):
    i = pl.program_id(0)
    m = m_ref[0] + m_ref[1] + h_ref[...]
    deg = degw_ref[0, :, :1] + degw_ref[1, :, :1] + 1.0
    agg = m / deg
    h_new = jnp.maximum(
        jnp.dot(agg, w_ref[...], preferred_element_type=jnp.float32)
        + b_ref[...], 0.0)
    blk, d = h_new.shape
    gids = lax.broadcasted_iota(jnp.int32, (blk, g), 1)
    onehot = (batch_ref[...] == gids).astype(jnp.float32)

    @pl.when(i == 0)
    def _():
        sums_ref[...] = jnp.zeros_like(sums_ref)
        counts_ref[...] = jnp.zeros_like(counts_ref)
    contract = (((0,), (0,)), ((), ()))
    sums_ref[...] += lax.dot_general(
        onehot, h_new, contract, preferred_element_type=jnp.float32)
    counts_ref[...] += lax.dot_general(
        onehot, jnp.ones((blk, d), jnp.float32), contract,
        preferred_element_type=jnp.float32)


def _tc_layer3(m, h, degw, w, b, batch2d, g, blk):
    n, d = h.shape
    grid = (n // blk,)
    return pl.pallas_call(
        functools.partial(_tc_layer3_body, g),
        grid=grid,
        in_specs=[
            pl.BlockSpec((NC, blk, d), lambda i: (0, i, 0)),
            pl.BlockSpec((blk, d), lambda i: (i, 0)),
            pl.BlockSpec((NC, blk, 16), lambda i: (0, i, 0)),
            pl.BlockSpec((d, d), lambda i: (0, 0)),
            pl.BlockSpec((1, d), lambda i: (0, 0)),
            pl.BlockSpec((blk, 1), lambda i: (i, 0)),
        ],
        out_specs=[
            pl.BlockSpec((g, d), lambda i: (0, 0)),
            pl.BlockSpec((g, d), lambda i: (0, 0)),
        ],
        out_shape=[
            jax.ShapeDtypeStruct((g, d), jnp.float32),
            jax.ShapeDtypeStruct((g, d), jnp.float32),
        ],
    )(m, h, degw, w, b, batch2d)


def _tc_head_body(sums_ref, counts_ref, v1_ref, vb1_ref, v2r_ref, vb2_ref,
                  out_ref):
    pooled = sums_ref[...] / jnp.maximum(counts_ref[...], 1.0)
    hidden = jnp.maximum(
        jnp.dot(pooled, v1_ref[...], preferred_element_type=jnp.float32)
        + vb1_ref[...], 0.0)
    v = lax.dot_general(v2r_ref[...], hidden, (((1,), (1,)), ((), ())),
                        preferred_element_type=jnp.float32)
    out_ref[...] = v + vb2_ref[...]


def _tc_head(sums, counts, v1, vb1, v2r, vb2, g, d):
    return pl.pallas_call(
        _tc_head_body,
        out_shape=jax.ShapeDtypeStruct((1, g), jnp.float32),
    )(sums, counts, v1, vb1, v2r, vb2)


def kernel(x, edge_index, batch, W1, b1, W2, b2, W3, b3, V1, vb1, V2, vb2):
    n, d = x.shape
    e = edge_index.shape[1]
    g = 64
    nw = NC * NS
    nb = _round_up(_round_up(e, nw * EB) // (nw * EB), 6)
    n_pad = _round_up(n + 1, NS * 8)
    e_pad = nw * nb * EB

    src_flat = jnp.concatenate(
        [edge_index[0], jnp.zeros((e_pad - e,), jnp.int32)])
    dst_flat = jnp.concatenate(
        [edge_index[1], jnp.full((e_pad - e,), n, jnp.int32)])
    dst = dst_flat.reshape(nw, nb, EB)

    nb0 = int(round(2 * nb * SPLIT0 / 6.0)) * 6
    nb1 = 2 * nb - nb0
    nbm = max(nb0, nb1)
    cut = NS * nb0 * EB
    s0 = src_flat[:cut].reshape(NS, nb0, EB)
    d0 = dst_flat[:cut].reshape(NS, nb0, EB)
    s1 = src_flat[cut:].reshape(NS, nb1, EB)
    d1 = dst_flat[cut:].reshape(NS, nb1, EB)
    if nb0 < nbm:
        padb = jnp.zeros((NS, nbm - nb0, EB), jnp.int32)
        s0 = jnp.concatenate([s0, padb], axis=1)
        d0 = jnp.concatenate([d0, padb + n], axis=1)
    if nb1 < nbm:
        padb = jnp.zeros((NS, nbm - nb1, EB), jnp.int32)
        s1 = jnp.concatenate([s1, padb], axis=1)
        d1 = jnp.concatenate([d1, padb + n], axis=1)
    src_a = jnp.concatenate([s0, s1], axis=0)
    dst_a = jnp.concatenate([d0, d1], axis=0)

    sc_msg = _make_sc_msg(n_pad, nb0, nb1, d)
    sc_deg = _make_sc_deg(n_pad, nb, d)

    degw = sc_deg(dst)[:, :, :16]

    blk = 1000
    b1r = b1.reshape(1, d)
    b2r = b2.reshape(1, d)
    b3r = b3.reshape(1, d)
    batch2d = batch.reshape(n, 1)

    m1 = sc_msg(x, src_a, dst_a)
    h1 = _tc_layer(m1, x, degw, W1, b1r, blk)
    m2 = sc_msg(h1, src_a, dst_a)
    h2 = _tc_layer(m2, h1, degw, W2, b2r, blk)
    m3 = sc_msg(h2, src_a, dst_a)
    sums, counts = _tc_layer3(m3, h2, degw, W3, b3r, batch2d, g, blk)

    v = _tc_head(sums, counts, V1, vb1.reshape(1, d),
                 V2.reshape(1, d), vb2.reshape(1, 1), g, d)
    return v.reshape(g)
